# baseline (speedup 1.0000x reference)
import functools

import jax
import jax.numpy as jnp
from jax import lax
from jax.experimental import pallas as pl
from jax.experimental.pallas import tpu as pltpu
from jax.experimental.pallas import tpu_sc as plsc


def kernel(data, offsets, weight):
    NNZ = data.shape[0]
    B = offsets.shape[0] - 1
    NE, D = weight.shape
    NC, NS = 2, 16
    NW = NC * NS
    BPW = B // NW
    BPC = B // NC
    L = 16
    C = 512
    RB = 65536
    SUB = 128
    TAILW = 128
    SEGM = 8192
    DUMMY = BPC
    RPAD = NNZ + 2048 + SEGM + 512
    TRASH0 = NNZ + 2048 + SEGM

    wT = weight.T
    wtailT = lax.slice(wT, (0, NE - TAILW), (D, NE))

    mesh = plsc.VectorSubcoreMesh(core_axis_name="c", subcore_axis_name="s")

    @functools.partial(
        pl.kernel,
        out_type=jax.ShapeDtypeStruct((B, D), jnp.float32),
        mesh=mesh,
        scratch_types=[
            pltpu.VMEM((BPW + 16,), jnp.int32),
            pltpu.VMEM((C,), jnp.int32),
            pltpu.VMEM((C,), jnp.int32),
            pltpu.VMEM((4, 128), jnp.int32),
            pltpu.VMEM((C,), jnp.int32),
            pltpu.VMEM((16,), jnp.int32),
            pltpu.VMEM((16,), jnp.int32),
            pltpu.VMEM((256,), jnp.int32),
            pltpu.VMEM((16,), jnp.int32),
            pltpu.VMEM((SEGM + 16,), jnp.int32),
            pltpu.VMEM((SEGM + 16,), jnp.int32),
            pltpu.VMEM((528,), jnp.int32),
            pltpu.VMEM((528,), jnp.int32),
            pltpu.VMEM((528,), jnp.int32),
            pltpu.VMEM((D, SUB), jnp.float32),
            pltpu.VMEM((256, D), jnp.float32),
            pltpu.VMEM((2, 128), jnp.int32),
            pltpu.VMEM_SHARED((RPAD,), jnp.int32),
            pltpu.VMEM_SHARED((256,), jnp.int32),
            pltpu.VMEM_SHARED((NS * 513, D), jnp.float32),
        ],
        compiler_params=pltpu.CompilerParams(
            needs_layout_passes=False, use_tc_tiling_on_sc=False),
    )
    def emb_bag(data_h, offs_h, wt_h, wtail_h, out_h,
                off_v, dbuf, hist_v, pbuf, rbuf, cnts_v, ccnt_v,
                cntm_v, tails_v, r2, r3, st2, tl2, cnt2,
                stage_v, rows_pick, segw, rec_sh, cnt_sh, acc_sh):
        cid = lax.axis_index("c")
        sid = lax.axis_index("s")
        w = cid * NS + sid
        b0 = w * BPW
        iota = lax.iota(jnp.int32, L)
        zi = jnp.zeros((L,), jnp.int32)
        zf = jnp.zeros((L,), jnp.float32)
        ones = jnp.ones((L,), jnp.int32)
        trash = TRASH0 + sid * L + iota

        pltpu.sync_copy(offs_h.at[pl.ds(b0, BPW + 1)],
                        off_v.at[pl.ds(0, BPW + 1)])
        e0 = off_v[pl.ds(0, L)][0]
        e1 = jnp.where(w == NW - 1, NNZ, off_v[pl.ds(BPW, L)][0])
        base_a = (e0 // 8) * 8
        nchunks = (e1 - base_a + C - 1) // C

        def zrow(r, carry):
            for d4 in range(D // L):
                rows_pick[r, pl.ds(d4 * L, L)] = zf
            return carry
        lax.fori_loop(0, 256, zrow, 0)
        z0 = sid * 513
        pltpu.sync_copy(rows_pick, acc_sh.at[pl.ds(z0, 256)])
        pltpu.sync_copy(rows_pick, acc_sh.at[pl.ds(z0 + 256, 256)])
        pltpu.sync_copy(rows_pick.at[pl.ds(0, 1)],
                        acc_sh.at[pl.ds(z0 + 512, 1)])

        cnts_v[pl.ds(0, L)] = zi

        def count_chunk(kk, carry):
            s_k = base_a + kk * C
            base_k = jnp.minimum(s_k, NNZ - C)
            lo = jnp.maximum(s_k, e0)
            pltpu.sync_copy(data_h.at[pl.ds(base_k, C)], dbuf)
            ccnt_v[pl.ds(0, L)] = zi
            for j in range(C // L):
                rv = dbuf[pl.ds(j * L, L)]
                p = base_k + j * L + iota
                valid = (p >= lo) & (p < e1)
                bkt = jnp.minimum(lax.shift_right_logical(rv, 16), 15)
                plsc.addupdate_scatter(ccnt_v, [bkt], ones, mask=valid)
            cnts_v[pl.ds(0, L)] = cnts_v[pl.ds(0, L)] + ccnt_v[pl.ds(0, L)]
            return carry
        lax.fori_loop(0, nchunks, count_chunk, 0)

        pltpu.sync_copy(cnts_v,
                        cnt_sh.at[pl.ds(pl.multiple_of(sid * L, 8), L)])
        plsc.subcore_barrier()

        pltpu.sync_copy(cnt_sh, cntm_v)
        totals = zi
        myprefix = zi
        for t in range(NS):
            row = cntm_v[pl.ds(t * L, L)]
            rowr = ((row + 7) // 8) * 8
            totals = totals + rowr
            myprefix = myprefix + jnp.where(t < sid, rowr, zi)
        bucket_base = plsc.cumsum(totals) - totals
        wstart = bucket_base + myprefix
        tails_v[pl.ds(0, L)] = wstart
        myrow = cnts_v[pl.ds(0, L)]
        rend = wstart + ((myrow + 7) // 8) * 8
        n_mine = jnp.sum(jnp.where(iota == sid, totals, zi))
        reg0 = jnp.sum(jnp.where(iota == sid, bucket_base, zi))

        def rec_chunk(kk, run):
            s_k = base_a + kk * C
            base_k = jnp.minimum(s_k, NNZ - C)
            lo = jnp.maximum(s_k, e0)
            pltpu.sync_copy(data_h.at[pl.ds(base_k, C)], dbuf)
            for j in range(C // L):
                hist_v[pl.ds(j * L, L)] = zi
            for j in range(BPW // L):
                ov = off_v[pl.ds(j * L, L)]
                m = (ov >= s_k) & (ov - base_k < C)
                plsc.addupdate_scatter(hist_v, [ov - base_k], ones, mask=m)
            r = run
            for j in range(C // L):
                h = hist_v[pl.ds(j * L, L)]
                cs = plsc.cumsum(h)
                p = base_k + j * L + iota
                valid = (p >= lo) & (p < e1)
                seg = jnp.where(valid, sid * BPW + r + cs - 1, DUMMY)
                rv = dbuf[pl.ds(j * L, L)]
                bkt = jnp.minimum(lax.shift_right_logical(rv, 16), 15)
                rloc = rv - lax.shift_left(bkt, 16)
                pack = lax.bitwise_or(lax.shift_left(rloc, 14), seg)
                cnt, last = plsc.scan_count(bkt, mask=valid)
                t16 = plsc.load_gather(tails_v, [bkt])
                pos = jnp.where(valid, t16 + cnt - 1, trash)
                plsc.addupdate_scatter(tails_v, [bkt], cnt,
                                       mask=last & valid)
                pbuf[j // 8, pl.ds((j % 8) * L, L)] = pos
                rbuf[pl.ds(j * L, L)] = pack
                r = r + jnp.sum(h)
            for q in range(C // 128):
                pltpu.sync_copy(rbuf.at[pl.ds(q * 128, 128)],
                                rec_sh.at[pbuf.at[q]])
            return r
        lax.fori_loop(0, nchunks, rec_chunk, jnp.int32(0))

        padrec = jnp.full((L,), DUMMY, jnp.int32)
        for wave in range(8):
            tcur = tails_v[pl.ds(0, L)]
            active = tcur < rend
            pos = jnp.where(active, tcur, trash)
            pbuf[0, pl.ds(0, L)] = pos
            rbuf[pl.ds(0, L)] = padrec
            pltpu.sync_copy(rbuf.at[pl.ds(0, L)],
                            rec_sh.at[pbuf.at[0, pl.ds(0, L)]])
            tails_v[pl.ds(0, L)] = tcur + jnp.where(active, ones, zi)
        plsc.subcore_barrier()

        bucket_r0 = sid * RB
        n_sub = jnp.where(sid == NS - 1, 133, RB // SUB)
        nseg = (n_mine + SEGM - 1) // SEGM

        def seg_loop(seg_i, fill):
            seg_base = pl.multiple_of(reg0 + seg_i * SEGM, 8)
            m_seg = jnp.minimum(SEGM, n_mine - seg_i * SEGM)
            pltpu.sync_copy(rec_sh.at[pl.ds(seg_base, SEGM)],
                            r2.at[pl.ds(0, SEGM)])
            ngrp = (m_seg + L - 1) // L

            for j in range(528 // L):
                cnt2[pl.ds(j * L, L)] = zi

            def cpass(g, carry):
                pk = r2[pl.ds(g * L, L)]
                gv = (g * L + iota) < m_seg
                sub = lax.shift_right_logical(pk, 21)
                sub = jnp.where(gv, sub, 0)
                plsc.addupdate_scatter(cnt2, [sub], ones, mask=gv)
                return carry
            lax.fori_loop(0, ngrp, cpass, 0)

            run = jnp.int32(0)
            for j in range(528 // L):
                h = cnt2[pl.ds(j * L, L)]
                cs = plsc.cumsum(h)
                st2[pl.ds(j * L, L)] = run + cs - h
                tl2[pl.ds(j * L, L)] = run + cs - h
                run = run + jnp.sum(h)
            st2[pl.ds(512, L)] = jnp.full((L,), run, jnp.int32)

            def zr3(g, carry):
                r3[pl.ds(g * L, L)] = zi
                return carry
            lax.fori_loop(0, (SEGM + 16) // L, zr3, 0)

            def spass(g, carry):
                pk = r2[pl.ds(g * L, L)]
                gv = (g * L + iota) < m_seg
                sub = lax.shift_right_logical(pk, 21)
                sub = jnp.where(gv, sub, 0)
                cnt, last = plsc.scan_count(sub, mask=gv)
                t16 = plsc.load_gather(tl2, [sub])
                pos = jnp.where(gv, jnp.clip(t16 + cnt - 1, 0, SEGM - 1),
                                SEGM + iota)
                plsc.addupdate_scatter(r3, [pos], pk, mask=gv)
                plsc.addupdate_scatter(tl2, [sub], cnt, mask=last & gv)
                return carry
            lax.fori_loop(0, ngrp, spass, 0)

            def sub_loop(s, fill_in):
                is_tail = (s == 132) & (sid == NS - 1)

                @pl.when(jnp.logical_not(is_tail))
                def _():
                    pltpu.sync_copy(
                        wt_h.at[:, pl.ds(bucket_r0 + s * SUB, SUB)], stage_v)

                @pl.when(is_tail)
                def _():
                    pltpu.sync_copy(wtail_h, stage_v)
                sub_base = jnp.where(is_tail, NE - TAILW - bucket_r0,
                                     s * SUB)
                stv = st2[pl.ds(s, L)]
                st = stv[0]
                en = stv[1]
                ng = (en - st + L - 1) // L

                def pick_grp(g, fi):
                    e = st + g * L
                    pk = r3[pl.ds(e, L)]
                    gm = (e + iota) < en
                    rl16 = jnp.where(
                        gm, lax.shift_right_logical(pk, 14) - sub_base, 0)
                    svm = jnp.where(gm, lax.bitwise_and(pk, 16383), DUMMY)
                    row0 = lax.rem(fi, 256)
                    grp = lax.rem(fi // 128, 2)
                    segw[grp, pl.ds(lax.rem(fi, 128), L)] = svm
                    for i in range(L):
                        rlv = jnp.full((L,), rl16[i], jnp.int32)
                        for kq in range(D // L):
                            pick = plsc.load_gather(
                                stage_v, [kq * L + iota, rlv])
                            rows_pick[row0 + i, pl.ds(kq * L, L)] = pick
                    fi = fi + L

                    @pl.when(lax.rem(fi, 128) == 0)
                    def _():
                        fgrp = lax.rem((fi - 128) // 128, 2)
                        pltpu.sync_copy(
                            rows_pick.at[pl.ds(lax.rem(fi - 128, 256), 128)],
                            acc_sh.at[segw.at[fgrp]], add=True)
                    return fi
                return lax.fori_loop(0, ng, pick_grp, fill_in)
            return lax.fori_loop(0, n_sub, sub_loop, fill)
        fill = lax.fori_loop(0, nseg, seg_loop, jnp.int32(0))

        @pl.when(lax.rem(fill, 128) != 0)
        def _():
            def padg(g, fi):
                grp = lax.rem(fi // 128, 2)
                segw[grp, pl.ds(lax.rem(fi, 128), L)] = \
                    jnp.full((L,), DUMMY, jnp.int32)
                return fi + L
            nmiss = (128 - lax.rem(fill, 128)) // L
            fend = lax.fori_loop(0, nmiss, padg, fill)
            fgrp = lax.rem((fend - 128) // 128, 2)
            pltpu.sync_copy(
                rows_pick.at[pl.ds(lax.rem(fend - 128, 256), 128)],
                acc_sh.at[segw.at[fgrp]], add=True)

        plsc.subcore_barrier()

        for half in range(2):
            pltpu.sync_copy(
                acc_sh.at[pl.ds(sid * BPW + half * 256, 256)], rows_pick)

            def fin(g, carry):
                bb = half * 256 + g * L
                o0v = off_v[pl.ds(bb, L)]
                o1v = off_v[pl.ds(bb + 1, L)]
                o1v = jnp.where(bb + iota == BPW - 1, e1, o1v)
                cntv = (o1v - o0v).astype(jnp.float32)
                scv = jnp.where(cntv > 0.0, 1.0 / cntv, 0.0)
                for i in range(L):
                    sv = jnp.full((L,), scv[i], jnp.float32)
                    for d4 in range(D // L):
                        rows_pick[g * L + i, pl.ds(d4 * L, L)] = (
                            rows_pick[g * L + i, pl.ds(d4 * L, L)] * sv)
                return carry
            lax.fori_loop(0, 256 // L, fin, 0)
            pltpu.sync_copy(rows_pick,
                            out_h.at[pl.ds(b0 + half * 256, 256)])

    return emb_bag(data, offsets, wT, wtailT)


# speedup vs baseline: 8.7988x; 8.7988x over previous
import functools

import jax
import jax.numpy as jnp
from jax import lax
from jax.experimental import pallas as pl
from jax.experimental.pallas import tpu as pltpu
from jax.experimental.pallas import tpu_sc as plsc


def kernel(data, offsets, weight):
    NNZ = data.shape[0]
    B = offsets.shape[0] - 1
    NE, D = weight.shape
    NC, NS = 2, 16
    NW = NC * NS
    BPW = B // NW
    C = 512
    NQ = C // 128
    SLAB = BPW + 1
    L = 16

    mesh = plsc.VectorSubcoreMesh(core_axis_name="c", subcore_axis_name="s")

    @functools.partial(
        pl.kernel,
        out_type=jax.ShapeDtypeStruct((B, D), jnp.float32),
        mesh=mesh,
        scratch_types=[
            pltpu.VMEM((BPW + 16,), jnp.int32),
            pltpu.VMEM((C,), jnp.int32),
            pltpu.VMEM((NQ, 128), jnp.int32),
            pltpu.VMEM((C,), jnp.int32),
            pltpu.VMEM((C, D), jnp.float32),
            pltpu.VMEM_SHARED((NS * SLAB, D), jnp.float32),
            pltpu.SemaphoreType.DMA,
        ],
        compiler_params=pltpu.CompilerParams(
            needs_layout_passes=False, use_tc_tiling_on_sc=False),
    )
    def emb_bag(data_h, offs_h, weight_h, out_h,
                off_v, idx_v, seg_v, hist_v, rows_v, acc_sh, sem):
        cid = lax.axis_index("c")
        sid = lax.axis_index("s")
        w = cid * NS + sid
        b0 = w * BPW
        slab0 = sid * SLAB
        dummy = slab0 + BPW

        pltpu.sync_copy(offs_h.at[pl.ds(b0, BPW + 1)],
                        off_v.at[pl.ds(0, BPW + 1)])
        e0 = off_v[pl.ds(0, L)][0]
        e1 = jnp.where(w == NW - 1, NNZ, off_v[pl.ds(BPW, L)][0])
        base_a = (e0 // 8) * 8

        zf = jnp.zeros((L,), jnp.float32)
        zi = jnp.zeros((L,), jnp.int32)
        ones = jnp.ones((L,), jnp.int32)
        iota = lax.iota(jnp.int32, L)

        def zrow(r, carry):
            for d in range(D // L):
                rows_v[r, pl.ds(d * L, L)] = zf
            return carry
        lax.fori_loop(0, C, zrow, 0)
        pltpu.sync_copy(rows_v, acc_sh.at[pl.ds(slab0, C)])
        pltpu.sync_copy(rows_v.at[pl.ds(0, 1)],
                        acc_sh.at[pl.ds(slab0 + BPW, 1)])

        nchunks = (e1 - base_a + C - 1) // C

        def chunk(kk, run):
            s_k = base_a + kk * C
            base_k = jnp.minimum(s_k, NNZ - C)
            pltpu.sync_copy(data_h.at[pl.ds(base_k, C)], idx_v)
            cps = [
                pltpu.async_copy(
                    weight_h.at[idx_v.at[pl.ds(q * 128, 128)]],
                    rows_v.at[pl.ds(q * 128, 128)], sem)
                for q in range(NQ)
            ]
            for j in range(C // L):
                hist_v[pl.ds(j * L, L)] = zi
            for j in range(BPW // L):
                ov = off_v[pl.ds(j * L, L)]
                m = (ov >= s_k) & (ov - base_k < C)
                plsc.addupdate_scatter(hist_v, [ov - base_k], ones, mask=m)
            lo = jnp.maximum(s_k, e0)
            r = run
            for j in range(C // L):
                h = hist_v[pl.ds(j * L, L)]
                cs = plsc.cumsum(h)
                p = base_k + j * L + iota
                valid = (p >= lo) & (p < e1)
                seg = jnp.where(valid, slab0 + r + cs - 1, dummy)
                seg_v[j // 8, pl.ds((j % 8) * L, L)] = seg
                r = r + jnp.sum(h)
            for cp in cps:
                cp.wait()
            for q in range(NQ):
                pltpu.sync_copy(rows_v.at[pl.ds(q * 128, 128)],
                                acc_sh.at[seg_v.at[q]], add=True)
            return r

        lax.fori_loop(0, nchunks, chunk, jnp.int32(0))

        pltpu.sync_copy(acc_sh.at[pl.ds(slab0, BPW)], rows_v)

        def fin(g, carry):
            b = g * L
            o0v = off_v[pl.ds(b, L)]
            o1v = off_v[pl.ds(b + 1, L)]
            o1v = jnp.where(b + iota == BPW - 1, e1, o1v)
            cntv = (o1v - o0v).astype(jnp.float32)
            scv = jnp.where(cntv > 0.0, 1.0 / cntv, 0.0)
            for i in range(L):
                sv = jnp.full((L,), scv[i], jnp.float32)
                for d in range(D // L):
                    rows_v[b + i, pl.ds(d * L, L)] = (
                        rows_v[b + i, pl.ds(d * L, L)] * sv)
            return carry
        lax.fori_loop(0, BPW // L, fin, 0)

        pltpu.sync_copy(rows_v, out_h.at[pl.ds(b0, BPW)])

    return emb_bag(data, offsets, weight)
